# Initial kernel scaffold; baseline (speedup 1.0000x reference)
import jax
import jax.numpy as jnp
from jax.experimental import pallas as pl


def kernel(x, edge_index, batch, W1l, W1r, a1, b1, W2l, W2r, a2, b2, Wlin, blin):
    raise NotImplementedError("write your pallas kernel here")


# speedup vs baseline: 2.4792x; 2.4792x over previous
import jax
import jax.numpy as jnp
from jax.experimental import pallas as pl

N = 10000
G = 64


def _mm_body(x_ref, w_ref, o_ref):
    o_ref[...] = jnp.dot(x_ref[...], w_ref[...],
                         preferred_element_type=jnp.float32)


def _mm(x, w):
    return pl.pallas_call(
        _mm_body,
        out_shape=jax.ShapeDtypeStruct((x.shape[0], w.shape[1]), jnp.float32),
    )(x, w)


def _gat_layer(x, src, dst, Wl, Wr, att, bias):
    xl = _mm(x, Wl)
    xr = _mm(x, Wr)
    e = jax.nn.leaky_relu(xl[src] + xr[dst], negative_slope=0.2) @ att
    w = jnp.exp(e)
    denom = jax.ops.segment_sum(w, dst, num_segments=N)
    num = jax.ops.segment_sum(w[:, None] * xl[src], dst, num_segments=N)
    return num / (denom + 1e-16)[:, None] + bias


def kernel(x, edge_index, batch, W1l, W1r, a1, b1, W2l, W2r, a2, b2,
           Wlin, blin):
    src = edge_index[0]
    dst = edge_index[1]
    h = jax.nn.relu(_gat_layer(x, src, dst, W1l, W1r, a1, b1))
    h = jax.nn.relu(_gat_layer(h, src, dst, W2l, W2r, a2, b2))
    s = jax.ops.segment_sum(h, batch, num_segments=G)
    cnt = jax.ops.segment_sum(jnp.ones((N,), jnp.float32), batch,
                              num_segments=G)
    g = s / jnp.maximum(cnt, 1.0)[:, None]
    return _mm(g, Wlin) + blin
